# Initial kernel scaffold; baseline (speedup 1.0000x reference)
import jax
import jax.numpy as jnp
from jax.experimental import pallas as pl


def kernel(x, edge_index, W1, b1, W2, b2, W3, b3):
    raise NotImplementedError("write your pallas kernel here")


# speedup vs baseline: 17.5974x; 17.5974x over previous
import functools

import jax
import jax.numpy as jnp
from jax import lax
from jax.experimental import pallas as pl
from jax.experimental.pallas import tpu as pltpu
from jax.experimental.pallas import tpu_sc as plsc

N = 10000
E = 320000
DIN = 128
D = 64
NC, NS = 2, 16
NW = NC * NS
CH = 128
NCH = 79
EWP = NCH * CH
PADE = NW * EWP - E
N_ACC = 10112
RT = N_ACC // NS
DUMMY = N
DEGW = 16

_f32 = jnp.float32

_mesh = plsc.VectorSubcoreMesh(
    core_axis_name="c", subcore_axis_name="s", num_cores=NC, num_subcores=NS)


@functools.partial(
    pl.kernel,
    out_type=jax.ShapeDtypeStruct((NC, N_ACC, DEGW), _f32),
    mesh=_mesh,
    compiler_params=pltpu.CompilerParams(use_tc_tiling_on_sc=False),
    scratch_types=[
        pltpu.VMEM((NCH, CH), jnp.int32),
        pltpu.VMEM((CH, DEGW), _f32),
        pltpu.VMEM_SHARED((N_ACC, DEGW), _f32),
    ],
)
def _sc_degree(dst_hbm, ones_hbm, zeros_hbm, deg_hbm, idx_v, ones_v, deg_sh):
    c = lax.axis_index("c")
    s = lax.axis_index("s")
    w = s * NC + c
    r0 = s * RT
    pltpu.sync_copy(zeros_hbm.at[pl.ds(r0, RT)], deg_sh.at[pl.ds(r0, RT)])
    pltpu.sync_copy(dst_hbm.at[w], idx_v)
    pltpu.sync_copy(ones_hbm, ones_v)
    plsc.subcore_barrier()

    def body(j, carry):
        pltpu.sync_copy(ones_v, deg_sh.at[idx_v.at[j]], add=True)
        return carry

    lax.fori_loop(0, NCH, body, 0)
    plsc.subcore_barrier()
    pltpu.sync_copy(deg_sh.at[pl.ds(r0, RT)], deg_hbm.at[c, pl.ds(r0, RT)])


@functools.partial(
    pl.kernel,
    out_type=jax.ShapeDtypeStruct((NC, N_ACC, D), _f32),
    mesh=_mesh,
    compiler_params=pltpu.CompilerParams(use_tc_tiling_on_sc=False),
    scratch_types=[
        pltpu.VMEM((NCH, CH), jnp.int32),
        pltpu.VMEM((NCH, CH), jnp.int32),
        pltpu.VMEM((CH, D), _f32),
        pltpu.VMEM_SHARED((N_ACC, D), _f32),
    ],
)
def _sc_scatter(g_hbm, src_hbm, dst_hbm, zeros_hbm, out_hbm,
                idx_s, idx_d, rowbuf, s_sh):
    c = lax.axis_index("c")
    s = lax.axis_index("s")
    w = s * NC + c
    r0 = s * RT
    pltpu.sync_copy(zeros_hbm.at[pl.ds(r0, RT)], s_sh.at[pl.ds(r0, RT)])
    pltpu.sync_copy(src_hbm.at[w], idx_s)
    pltpu.sync_copy(dst_hbm.at[w], idx_d)
    plsc.subcore_barrier()

    def body(j, carry):
        pltpu.sync_copy(g_hbm.at[idx_s.at[j]], rowbuf)
        pltpu.sync_copy(rowbuf, s_sh.at[idx_d.at[j]], add=True)
        return carry

    lax.fori_loop(0, NCH, body, 0)
    plsc.subcore_barrier()
    pltpu.sync_copy(s_sh.at[pl.ds(r0, RT)], out_hbm.at[c, pl.ds(r0, RT)])


_BM = 1000


def _mm_body(x_ref, w_ref, o_ref):
    o_ref[...] = jnp.dot(x_ref[...], w_ref[...],
                         preferred_element_type=_f32)


_tc_matmul = pl.pallas_call(
    _mm_body,
    grid=(N // _BM,),
    in_specs=[
        pl.BlockSpec((_BM, DIN), lambda i: (i, 0)),
        pl.BlockSpec((DIN, D), lambda i: (0, 0)),
    ],
    out_specs=pl.BlockSpec((_BM, D), lambda i: (i, 0)),
    out_shape=jax.ShapeDtypeStruct((N, D), _f32),
)


def _scale_body(h_ref, deg_ref, g_ref, dis_ref):
    deg = deg_ref[0, :, 0:1] + deg_ref[1, :, 0:1] + 1.0
    dis = lax.rsqrt(deg)
    dis_ref[...] = dis
    g_ref[...] = dis * h_ref[...]


_tc_scale = pl.pallas_call(
    _scale_body,
    grid=(N // _BM,),
    in_specs=[
        pl.BlockSpec((_BM, D), lambda i: (i, 0)),
        pl.BlockSpec((NC, _BM, DEGW), lambda i: (0, i, 0)),
    ],
    out_specs=[
        pl.BlockSpec((_BM, D), lambda i: (i, 0)),
        pl.BlockSpec((_BM, 1), lambda i: (i, 0)),
    ],
    out_shape=[
        jax.ShapeDtypeStruct((N, D), _f32),
        jax.ShapeDtypeStruct((N, 1), _f32),
    ],
)


def _mid_body(s_ref, g_ref, dis_ref, b_ref, w_ref, o_ref):
    dis = dis_ref[...]
    z = jnp.maximum(dis * (s_ref[0] + s_ref[1] + g_ref[...]) + b_ref[...], 0.0)
    h2 = jnp.dot(z, w_ref[...], preferred_element_type=_f32)
    o_ref[...] = dis * h2


_tc_mid = pl.pallas_call(
    _mid_body,
    grid=(N // _BM,),
    in_specs=[
        pl.BlockSpec((NC, _BM, D), lambda i: (0, i, 0)),
        pl.BlockSpec((_BM, D), lambda i: (i, 0)),
        pl.BlockSpec((_BM, 1), lambda i: (i, 0)),
        pl.BlockSpec((1, D), lambda i: (0, 0)),
        pl.BlockSpec((D, D), lambda i: (0, 0)),
    ],
    out_specs=pl.BlockSpec((_BM, D), lambda i: (i, 0)),
    out_shape=jax.ShapeDtypeStruct((N, D), _f32),
)


def _out_body(s_ref, g_ref, dis_ref, b_ref, w_ref, b3_ref, o_ref):
    dis = dis_ref[...]
    z = jnp.maximum(dis * (s_ref[0] + s_ref[1] + g_ref[...]) + b_ref[...], 0.0)
    o_ref[...] = jnp.dot(z, w_ref[...], preferred_element_type=_f32) + b3_ref[...]


_tc_out = pl.pallas_call(
    _out_body,
    grid=(N // _BM,),
    in_specs=[
        pl.BlockSpec((NC, _BM, D), lambda i: (0, i, 0)),
        pl.BlockSpec((_BM, D), lambda i: (i, 0)),
        pl.BlockSpec((_BM, 1), lambda i: (i, 0)),
        pl.BlockSpec((1, D), lambda i: (0, 0)),
        pl.BlockSpec((D, 1), lambda i: (0, 0)),
        pl.BlockSpec((1, 1), lambda i: (0, 0)),
    ],
    out_specs=pl.BlockSpec((_BM, 1), lambda i: (i, 0)),
    out_shape=jax.ShapeDtypeStruct((N, 1), _f32),
)


def kernel(x, edge_index, W1, b1, W2, b2, W3, b3):
    ei = edge_index.astype(jnp.int32)
    src3 = jnp.concatenate(
        [ei[0], jnp.zeros((PADE,), jnp.int32)]).reshape(NW, NCH, CH)
    dst3 = jnp.concatenate(
        [ei[1], jnp.full((PADE,), DUMMY, jnp.int32)]).reshape(NW, NCH, CH)
    ones1 = jnp.ones((CH, DEGW), _f32)
    zeros1 = jnp.zeros((N_ACC, DEGW), _f32)
    zeros2 = jnp.zeros((N_ACC, D), _f32)
    b1r = b1.reshape(1, D)
    b2r = b2.reshape(1, D)
    b3r = b3.reshape(1, 1)

    degp = _sc_degree(dst3, ones1, zeros1)
    h1 = _tc_matmul(x, W1)
    g1, dis = _tc_scale(h1, degp)
    s1 = _sc_scatter(g1, src3, dst3, zeros2)
    g2 = _tc_mid(s1, g1, dis, b1r, W2)
    s2 = _sc_scatter(g2, src3, dst3, zeros2)
    return _tc_out(s2, g2, dis, b2r, W3, b3r)


# speedup vs baseline: 18.8245x; 1.0697x over previous
import functools

import jax
import jax.numpy as jnp
from jax import lax
from jax.experimental import pallas as pl
from jax.experimental.pallas import tpu as pltpu
from jax.experimental.pallas import tpu_sc as plsc

N = 10000
E = 320000
DIN = 128
D = 64
NC, NS = 2, 16
NW = NC * NS
CH = 128
NCH = 80
EWP = NCH * CH
PADE = NW * EWP - E
N_ACC = 10112
RT = N_ACC // NS
DUMMY = N
DEGW = 16

_f32 = jnp.float32

_mesh = plsc.VectorSubcoreMesh(
    core_axis_name="c", subcore_axis_name="s", num_cores=NC, num_subcores=NS)


@functools.partial(
    pl.kernel,
    out_type=jax.ShapeDtypeStruct((NC, N_ACC, DEGW), _f32),
    mesh=_mesh,
    compiler_params=pltpu.CompilerParams(use_tc_tiling_on_sc=False),
    scratch_types=[
        pltpu.VMEM((NCH, CH), jnp.int32),
        pltpu.VMEM((CH, DEGW), _f32),
        pltpu.VMEM_SHARED((N_ACC, DEGW), _f32),
    ],
)
def _sc_degree(dst_hbm, ones_hbm, zeros_hbm, deg_hbm, idx_v, ones_v, deg_sh):
    c = lax.axis_index("c")
    s = lax.axis_index("s")
    w = s * NC + c
    r0 = s * RT
    pltpu.sync_copy(zeros_hbm.at[pl.ds(r0, RT)], deg_sh.at[pl.ds(r0, RT)])
    pltpu.sync_copy(dst_hbm.at[w], idx_v)
    pltpu.sync_copy(ones_hbm, ones_v)
    plsc.subcore_barrier()

    def body(j, carry):
        pltpu.sync_copy(ones_v, deg_sh.at[idx_v.at[j]], add=True)
        return carry

    lax.fori_loop(0, NCH, body, 0)
    plsc.subcore_barrier()
    pltpu.sync_copy(deg_sh.at[pl.ds(r0, RT)], deg_hbm.at[c, pl.ds(r0, RT)])


@functools.partial(
    pl.kernel,
    out_type=jax.ShapeDtypeStruct((NC, N_ACC, D), _f32),
    mesh=_mesh,
    compiler_params=pltpu.CompilerParams(use_tc_tiling_on_sc=False),
    scratch_types=[
        pltpu.VMEM((NCH, CH), jnp.int32),
        pltpu.VMEM((NCH, CH), jnp.int32),
        pltpu.VMEM((2, CH, D), _f32),
        pltpu.VMEM_SHARED((N_ACC, D), _f32),
        pltpu.SemaphoreType.DMA,
        pltpu.SemaphoreType.DMA,
        pltpu.SemaphoreType.DMA,
        pltpu.SemaphoreType.DMA,
    ],
)
def _sc_scatter(g_hbm, src_hbm, dst_hbm, zeros_hbm, out_hbm,
                idx_s, idx_d, rowbuf, s_sh, gs0, gs1, ss0, ss1):
    c = lax.axis_index("c")
    s = lax.axis_index("s")
    w = s * NC + c
    r0 = s * RT
    pltpu.sync_copy(zeros_hbm.at[pl.ds(r0, RT)], s_sh.at[pl.ds(r0, RT)])
    pltpu.sync_copy(src_hbm.at[w], idx_s)
    pltpu.sync_copy(dst_hbm.at[w], idx_d)
    plsc.subcore_barrier()

    def wait_gather(j, b, sem):
        pltpu.make_async_copy(g_hbm.at[idx_s.at[j]], rowbuf.at[b], sem).wait()

    def wait_scatter(j, b, sem):
        pltpu.make_async_copy(rowbuf.at[b], s_sh.at[idx_d.at[j]], sem).wait()

    pltpu.async_copy(g_hbm.at[idx_s.at[0]], rowbuf.at[0], gs0)
    pltpu.async_copy(g_hbm.at[idx_s.at[1]], rowbuf.at[1], gs1)
    wait_gather(0, 0, gs0)
    pltpu.async_copy(rowbuf.at[0], s_sh.at[idx_d.at[0]], ss0, add=True)

    def outer(t, carry):
        j1 = 2 * t + 1
        wait_scatter(2 * t, 0, ss0)
        pltpu.async_copy(g_hbm.at[idx_s.at[j1 + 1]], rowbuf.at[0], gs0)
        wait_gather(j1, 1, gs1)
        pltpu.async_copy(rowbuf.at[1], s_sh.at[idx_d.at[j1]], ss1, add=True)
        wait_scatter(j1, 1, ss1)
        pltpu.async_copy(g_hbm.at[idx_s.at[j1 + 2]], rowbuf.at[1], gs1)
        wait_gather(j1 + 1, 0, gs0)
        pltpu.async_copy(rowbuf.at[0], s_sh.at[idx_d.at[j1 + 1]], ss0, add=True)
        return carry

    lax.fori_loop(0, NCH // 2 - 1, outer, 0)
    wait_scatter(NCH - 2, 0, ss0)
    wait_gather(NCH - 1, 1, gs1)
    pltpu.async_copy(rowbuf.at[1], s_sh.at[idx_d.at[NCH - 1]], ss1, add=True)
    wait_scatter(NCH - 1, 1, ss1)
    plsc.subcore_barrier()
    pltpu.sync_copy(s_sh.at[pl.ds(r0, RT)], out_hbm.at[c, pl.ds(r0, RT)])


_BM = 1000


def _mm_body(x_ref, w_ref, o_ref):
    o_ref[...] = jnp.dot(x_ref[...], w_ref[...],
                         preferred_element_type=_f32)


_tc_matmul = pl.pallas_call(
    _mm_body,
    grid=(N // _BM,),
    in_specs=[
        pl.BlockSpec((_BM, DIN), lambda i: (i, 0)),
        pl.BlockSpec((DIN, D), lambda i: (0, 0)),
    ],
    out_specs=pl.BlockSpec((_BM, D), lambda i: (i, 0)),
    out_shape=jax.ShapeDtypeStruct((N, D), _f32),
)


def _scale_body(h_ref, deg_ref, g_ref, dis_ref):
    deg = deg_ref[0, :, 0:1] + deg_ref[1, :, 0:1] + 1.0
    dis = lax.rsqrt(deg)
    dis_ref[...] = dis
    g_ref[...] = dis * h_ref[...]


_tc_scale = pl.pallas_call(
    _scale_body,
    grid=(N // _BM,),
    in_specs=[
        pl.BlockSpec((_BM, D), lambda i: (i, 0)),
        pl.BlockSpec((NC, _BM, DEGW), lambda i: (0, i, 0)),
    ],
    out_specs=[
        pl.BlockSpec((_BM, D), lambda i: (i, 0)),
        pl.BlockSpec((_BM, 1), lambda i: (i, 0)),
    ],
    out_shape=[
        jax.ShapeDtypeStruct((N, D), _f32),
        jax.ShapeDtypeStruct((N, 1), _f32),
    ],
)


def _mid_body(s_ref, g_ref, dis_ref, b_ref, w_ref, o_ref):
    dis = dis_ref[...]
    z = jnp.maximum(dis * (s_ref[0] + s_ref[1] + g_ref[...]) + b_ref[...], 0.0)
    h2 = jnp.dot(z, w_ref[...], preferred_element_type=_f32)
    o_ref[...] = dis * h2


_tc_mid = pl.pallas_call(
    _mid_body,
    grid=(N // _BM,),
    in_specs=[
        pl.BlockSpec((NC, _BM, D), lambda i: (0, i, 0)),
        pl.BlockSpec((_BM, D), lambda i: (i, 0)),
        pl.BlockSpec((_BM, 1), lambda i: (i, 0)),
        pl.BlockSpec((1, D), lambda i: (0, 0)),
        pl.BlockSpec((D, D), lambda i: (0, 0)),
    ],
    out_specs=pl.BlockSpec((_BM, D), lambda i: (i, 0)),
    out_shape=jax.ShapeDtypeStruct((N, D), _f32),
)


def _out_body(s_ref, g_ref, dis_ref, b_ref, w_ref, b3_ref, o_ref):
    dis = dis_ref[...]
    z = jnp.maximum(dis * (s_ref[0] + s_ref[1] + g_ref[...]) + b_ref[...], 0.0)
    o_ref[...] = jnp.dot(z, w_ref[...], preferred_element_type=_f32) + b3_ref[...]


_tc_out = pl.pallas_call(
    _out_body,
    grid=(N // _BM,),
    in_specs=[
        pl.BlockSpec((NC, _BM, D), lambda i: (0, i, 0)),
        pl.BlockSpec((_BM, D), lambda i: (i, 0)),
        pl.BlockSpec((_BM, 1), lambda i: (i, 0)),
        pl.BlockSpec((1, D), lambda i: (0, 0)),
        pl.BlockSpec((D, 1), lambda i: (0, 0)),
        pl.BlockSpec((1, 1), lambda i: (0, 0)),
    ],
    out_specs=pl.BlockSpec((_BM, 1), lambda i: (i, 0)),
    out_shape=jax.ShapeDtypeStruct((N, 1), _f32),
)


def kernel(x, edge_index, W1, b1, W2, b2, W3, b3):
    ei = edge_index.astype(jnp.int32)
    src3 = jnp.concatenate(
        [ei[0], jnp.zeros((PADE,), jnp.int32)]).reshape(NW, NCH, CH)
    dst3 = jnp.concatenate(
        [ei[1], jnp.full((PADE,), DUMMY, jnp.int32)]).reshape(NW, NCH, CH)
    ones1 = jnp.ones((CH, DEGW), _f32)
    zeros1 = jnp.zeros((N_ACC, DEGW), _f32)
    zeros2 = jnp.zeros((N_ACC, D), _f32)
    b1r = b1.reshape(1, D)
    b2r = b2.reshape(1, D)
    b3r = b3.reshape(1, 1)

    degp = _sc_degree(dst3, ones1, zeros1)
    h1 = _tc_matmul(x, W1)
    g1, dis = _tc_scale(h1, degp)
    s1 = _sc_scatter(g1, src3, dst3, zeros2)
    g2 = _tc_mid(s1, g1, dis, b1r, W2)
    s2 = _sc_scatter(g2, src3, dst3, zeros2)
    return _tc_out(s2, g2, dis, b2r, W3, b3r)
